# baseline (speedup 1.0000x reference)
import functools

import jax
import jax.numpy as jnp
from jax import lax
from jax.experimental import pallas as pl
from jax.experimental.pallas import tpu as pltpu
from jax.experimental.pallas import tpu_sc as plsc

_NUM_CORES = 2
_NUM_SUBCORES = 16
_NW = _NUM_CORES * _NUM_SUBCORES
_SLOT_ROWS = 4
_NBUF = 4


def _gather_body(nslot, n_tok, idx_hbm, table_hbm, out_hbm, idx_v, *scratch):
    slots = scratch[:_NBUF]
    gsem = scratch[_NBUF:2 * _NBUF]
    osem = scratch[2 * _NBUF:3 * _NBUF]
    wid = lax.axis_index("s") * _NUM_CORES + lax.axis_index("c")
    pltpu.sync_copy(idx_hbm.at[wid], idx_v)
    base_row = wid * nslot * _SLOT_ROWS

    def fire_gathers(j, b):
        for r in range(_SLOT_ROWS):
            pltpu.async_copy(table_hbm.at[idx_v.at[j * _SLOT_ROWS + r]],
                             slots[b].at[r], gsem[b])

    def wait_gathers(j, b):
        for r in range(_SLOT_ROWS):
            pltpu.make_async_copy(table_hbm.at[idx_v.at[j * _SLOT_ROWS + r]],
                                  slots[b].at[r], gsem[b]).wait()

    def o_descr(j, b):
        off = pl.multiple_of(base_row + j * _SLOT_ROWS, _SLOT_ROWS)
        return slots[b], out_hbm.at[pl.ds(off, _SLOT_ROWS)]

    def fire_out(j, b):
        src, dst = o_descr(j, b)
        pltpu.async_copy(src, dst, osem[b])

    def wait_out(j, b):
        src, dst = o_descr(j, b)
        pltpu.make_async_copy(src, dst, osem[b]).wait()

    for b in range(_NBUF - 1):
        fire_gathers(b, b)

    def step(i, carry):
        for b in range(_NBUF):
            j = i * _NBUF + b
            bp = (b - 1) % _NBUF
            jn = j + _NBUF - 1

            @pl.when(j > 0)
            def _():
                wait_out(j - 1, bp)

            @pl.when(jn < nslot)
            def _():
                fire_gathers(jn, bp)

            wait_gathers(j, b)
            fire_out(j, b)
        return carry

    lax.fori_loop(0, nslot // _NBUF, step, 0)
    wait_out(nslot - 1, (nslot - 1) % _NBUF)


@functools.partial(jax.jit, static_argnums=(2, 3, 4))
def _gather(idx, table, n_batch, n_tok, d):
    nslot = n_batch // _NW // _SLOT_ROWS
    mesh = plsc.VectorSubcoreMesh(core_axis_name="c", subcore_axis_name="s")
    f = pl.kernel(
        functools.partial(_gather_body, nslot, n_tok),
        out_type=jax.ShapeDtypeStruct((n_batch, n_tok, d), jnp.float32),
        mesh=mesh,
        scratch_types=(
            [pltpu.VMEM((nslot * _SLOT_ROWS, n_tok), jnp.int32)]
            + [pltpu.VMEM((_SLOT_ROWS, n_tok, d), jnp.float32)] * _NBUF
            + [pltpu.SemaphoreType.DMA] * (2 * _NBUF)
        ),
    )
    return f(idx, table)


def kernel(token_ids, embedding):
    n_batch, n_tok = token_ids.shape
    d = embedding.shape[1]
    idx = token_ids.reshape(_NW, n_batch // _NW, n_tok).astype(jnp.int32)
    return _gather(idx, embedding, n_batch, n_tok, d)


# speedup vs baseline: 1.0119x; 1.0119x over previous
import functools

import jax
import jax.numpy as jnp
from jax import lax
from jax.experimental import pallas as pl
from jax.experimental.pallas import tpu as pltpu
from jax.experimental.pallas import tpu_sc as plsc

_NUM_CORES = 2
_NUM_SUBCORES = 16
_NW = _NUM_CORES * _NUM_SUBCORES
_IPS = 100
_NBUF = 8


def _gather_body(nstream, n_tok, idx_hbm, table_hbm, out_hbm, idx_v, *scratch):
    slots = scratch[:_NBUF]
    gsem = scratch[_NBUF:2 * _NBUF]
    osem = scratch[2 * _NBUF:3 * _NBUF]
    wid = lax.axis_index("s") * _NUM_CORES + lax.axis_index("c")
    pltpu.sync_copy(idx_hbm.at[wid], idx_v)
    rows_per_stream = _IPS // n_tok
    base_row = wid * nstream * rows_per_stream

    def g_descr(j, b):
        return table_hbm.at[idx_v.at[j]], slots[b]

    def o_descr(j, r, b):
        src = slots[b].at[pl.ds(r * n_tok, n_tok)]
        dst = out_hbm.at[base_row + j * rows_per_stream + r]
        return src, dst

    def fire_gather(j, b):
        src, dst = g_descr(j, b)
        pltpu.async_copy(src, dst, gsem[b])

    def wait_gather(j, b):
        src, dst = g_descr(j, b)
        pltpu.make_async_copy(src, dst, gsem[b]).wait()

    def fire_outs(j, b):
        for r in range(rows_per_stream):
            src, dst = o_descr(j, r, b)
            pltpu.async_copy(src, dst, osem[b])

    def wait_outs(j, b):
        for r in range(rows_per_stream):
            src, dst = o_descr(j, r, b)
            pltpu.make_async_copy(src, dst, osem[b]).wait()

    for b in range(_NBUF - 1):
        fire_gather(b, b)

    def step(i, carry):
        for b in range(_NBUF):
            j = i * _NBUF + b
            bp = (b - 1) % _NBUF
            jn = j + _NBUF - 1

            @pl.when(j > 0)
            def _():
                wait_outs(j - 1, bp)

            @pl.when(jn < nstream)
            def _():
                fire_gather(jn, bp)

            wait_gather(j, b)
            fire_outs(j, b)
        return carry

    lax.fori_loop(0, nstream // _NBUF, step, 0)
    wait_outs(nstream - 1, (nstream - 1) % _NBUF)


@functools.partial(jax.jit, static_argnums=(2, 3, 4))
def _gather(idx, table, n_batch, n_tok, d):
    nstream = n_batch * n_tok // _NW // _IPS
    mesh = plsc.VectorSubcoreMesh(core_axis_name="c", subcore_axis_name="s")
    f = pl.kernel(
        functools.partial(_gather_body, nstream, n_tok),
        out_type=jax.ShapeDtypeStruct((n_batch, n_tok, d), jnp.float32),
        mesh=mesh,
        scratch_types=(
            [pltpu.VMEM((nstream, _IPS), jnp.int32)]
            + [pltpu.VMEM((_IPS, d), jnp.float32)] * _NBUF
            + [pltpu.SemaphoreType.DMA] * (2 * _NBUF)
        ),
    )
    return f(idx, table)


def kernel(token_ids, embedding):
    n_batch, n_tok = token_ids.shape
    d = embedding.shape[1]
    nstream = n_batch * n_tok // _NW // _IPS
    idx = token_ids.reshape(_NW, nstream, _IPS).astype(jnp.int32)
    return _gather(idx, embedding, n_batch, n_tok, d)
